# baseline (speedup 1.0000x reference)
import functools

import jax
import jax.numpy as jnp
from jax import lax
from jax.experimental import pallas as pl
from jax.experimental.pallas import tpu as pltpu
from jax.experimental.pallas import tpu_sc as plsc

B = 4096
L = 50
VID_BUCKETS = 1000000
AUT_BUCKETS = 100000

NW = 32
CHUNK = 128


def _hash16(x, num_buckets):
    return jnp.where(x == 0, 0, lax.rem(x, num_buckets - 1) + 1)


@functools.lru_cache(maxsize=1)
def _build_sc_gather():
    mesh = plsc.VectorSubcoreMesh(core_axis_name="c", subcore_axis_name="s")

    @functools.partial(
        pl.kernel,
        mesh=mesh,
        out_type=[
            jax.ShapeDtypeStruct((B, 128), jnp.float32),
            jax.ShapeDtypeStruct((L * B, 128), jnp.float32),
            jax.ShapeDtypeStruct((L, B), jnp.int32),
        ],
        scratch_types=[
            pltpu.VMEM((CHUNK, L), jnp.int32),
            pltpu.VMEM((CHUNK, L), jnp.int32),
            pltpu.VMEM((CHUNK, L), jnp.int32),
            pltpu.VMEM((CHUNK,), jnp.int32),
            pltpu.VMEM((CHUNK,), jnp.int32),
            pltpu.VMEM((1, CHUNK), jnp.int32),
            pltpu.VMEM((CHUNK, 64), jnp.float32),
            pltpu.VMEM((CHUNK, 64), jnp.float32),
            pltpu.SemaphoreType.DMA,
            pltpu.SemaphoreType.DMA,
        ],
        compiler_params=pltpu.CompilerParams(use_tc_tiling_on_sc=False,
                                             needs_layout_passes=False),
    )
    def _sc_gather(video_hbm, author_hbm, cvid_hbm, caid_hbm, hvid_hbm,
                   haid_hbm, mask_hbm, out_q, out_keys, out_maskT,
                   idxv2, idxa2, msk2, chv, cha, mcol, bufv, bufa,
                   semv, sema):
        _sc_gather_body(video_hbm, author_hbm, cvid_hbm, caid_hbm, hvid_hbm,
                        haid_hbm, mask_hbm, out_q, out_keys, out_maskT,
                        idxv2, idxa2, msk2, chv, cha, mcol, bufv, bufa,
                        semv, sema)

    return _sc_gather


def _sc_gather_body(video_hbm, author_hbm, cvid_hbm, caid_hbm, hvid_hbm,
                    haid_hbm, mask_hbm, out_q, out_keys, out_maskT,
                    idxv2, idxa2, msk2, chv, cha, mcol, bufv, bufa,
                    semv, sema):
    wid = lax.axis_index("s") * 2 + lax.axis_index("c")
    b0 = pl.multiple_of(wid * CHUNK, 8)

    pltpu.sync_copy(cvid_hbm.at[pl.ds(b0, CHUNK)], chv)
    pltpu.sync_copy(caid_hbm.at[pl.ds(b0, CHUNK)], cha)
    for r in range(CHUNK // 16):
        s = pl.ds(r * 16, 16)
        chv[s] = _hash16(chv[s], VID_BUCKETS)
        cha[s] = _hash16(cha[s], AUT_BUCKETS)
    cpv = pltpu.async_copy(video_hbm.at[chv], bufv, semv)
    cpa = pltpu.async_copy(author_hbm.at[cha], bufa, sema)
    cpv.wait()
    pltpu.sync_copy(bufv, out_q.at[pl.ds(b0, CHUNK), pl.ds(0, 64)])
    cpa.wait()
    pltpu.sync_copy(bufa, out_q.at[pl.ds(b0, CHUNK), pl.ds(64, 64)])

    pltpu.sync_copy(hvid_hbm.at[pl.ds(b0, CHUNK), :], idxv2)
    pltpu.sync_copy(haid_hbm.at[pl.ds(b0, CHUNK), :], idxa2)
    pltpu.sync_copy(mask_hbm.at[pl.ds(b0, CHUNK), :], msk2)

    def body(l, carry):
        lvec = jnp.full((16,), 0, jnp.int32) + l
        for c in range(CHUNK // 16):
            rows = lax.iota(jnp.int32, 16) + (c * 16)
            s = pl.ds(c * 16, 16)
            chv[s] = _hash16(plsc.load_gather(idxv2, [rows, lvec]), VID_BUCKETS)
            cha[s] = _hash16(plsc.load_gather(idxa2, [rows, lvec]), AUT_BUCKETS)
            mcol[0, s] = plsc.load_gather(msk2, [rows, lvec])
        cpv2 = pltpu.async_copy(video_hbm.at[chv], bufv, semv)
        cpa2 = pltpu.async_copy(author_hbm.at[cha], bufa, sema)
        rbase = pl.multiple_of(l * B + b0, 8)
        cpv2.wait()
        pltpu.sync_copy(bufv, out_keys.at[pl.ds(rbase, CHUNK), pl.ds(0, 64)])
        cpa2.wait()
        pltpu.sync_copy(bufa, out_keys.at[pl.ds(rbase, CHUNK), pl.ds(64, 64)])
        pltpu.sync_copy(mcol, out_maskT.at[pl.ds(l, 1), pl.ds(b0, CHUNK)])
        return carry

    lax.fori_loop(0, L, body, 0, unroll=False)


def _prelu(x, a):
    return jnp.where(x >= 0, x, a * x)


def _pad_rows(w, total):
    return jnp.concatenate(
        [w, jnp.zeros((total - w.shape[0], w.shape[1]), jnp.float32)], axis=0)


def _att_body(q_ref, keys_ref, mask_ref,
              W1_ref, b1_ref, a1_ref, W2_ref, b2_ref, a2_ref, W3_ref, b3_ref,
              out_ref):
    bB = q_ref.shape[0]
    q = q_ref[...]
    keys = keys_ref[...]
    mask = mask_ref[...]

    W1 = W1_ref[...]
    Wa = W1[0:96, :]
    Wb = W1[96:192, :]
    Wc = W1[192:288, :]
    Wd = W1[288:384, :]
    Wq = _pad_rows(Wa + Wc, 128)
    Wk = _pad_rows(Wb - Wc, 128)
    Wp = _pad_rows(Wd, 128)
    b1 = b1_ref[...]
    a1 = a1_ref[0, 0]
    W2 = W2_ref[...]
    b2 = b2_ref[...]
    a2 = a2_ref[0, 0]
    W3 = W3_ref[...]
    b3 = b3_ref[0, 0]

    dot = functools.partial(jnp.dot, preferred_element_type=jnp.float32)

    kf = keys.reshape(L * bB, 128)
    term_q = dot(q, Wq)
    term_k = dot(kf, Wk)
    pf = (keys * q[None, :, :]).reshape(L * bB, 128)
    term_p = dot(pf, Wp)

    h = term_k + term_p + jnp.broadcast_to(
        term_q[None, :, :], (L, bB, 80)).reshape(L * bB, 80)
    h = _prelu(h + b1, a1)
    h = _prelu(dot(h, W2) + b2, a2)
    scores = dot(h, W3).reshape(L, bB) + b3

    neg = jnp.float32(-10000.0)
    scores = jnp.where(mask == 0, neg, scores)
    m = jnp.max(scores, axis=0, keepdims=True)
    e = jnp.exp(scores - m)
    w = e / jnp.sum(e, axis=0, keepdims=True)
    w = jnp.where(mask == 0, jnp.float32(0.0), w)

    interest = jnp.sum(keys * w[:, :, None], axis=0)

    out_ref[...] = jnp.concatenate(
        [q[:, 0:96], interest[:, 0:96]], axis=1)


def _onehot_lookup(idx2d, table, n):
    oh = jnp.where(
        idx2d == lax.broadcasted_iota(jnp.int32, (idx2d.shape[0], n), 1),
        jnp.float32(1.0), jnp.float32(0.0))
    return jnp.dot(oh, table, preferred_element_type=jnp.float32)


def _bn_relu(x, g, bb):
    m = jnp.mean(x, axis=0, keepdims=True)
    v = jnp.mean((x - m) ** 2, axis=0, keepdims=True)
    return jnp.maximum(g * (x - m) / jnp.sqrt(v + 1e-5) + bb, 0.0)


def _dnn_body(qi_ref, vt_i_ref, tag_i_ref, tab_i_ref, uad_i_ref, fur_i_ref,
              vt_ref, tag_ref, tab_ref, uad_ref, fur_ref,
              D1_ref, db1_ref, g1_ref, bb1_ref,
              D2_ref, db2_ref, g2_ref, bb2_ref,
              D3_ref, db3_ref, g3_ref, bb3_ref,
              D4_ref, db4_ref, out_ref):
    qi = qi_ref[...]
    side = jnp.concatenate([
        _onehot_lookup(vt_i_ref[...], vt_ref[...], 5),
        _onehot_lookup(tag_i_ref[...], tag_ref[...], 80),
        _onehot_lookup(tab_i_ref[...], tab_ref[...], 10),
        _onehot_lookup(uad_i_ref[...], uad_ref[...], 8),
        _onehot_lookup(fur_i_ref[...], fur_ref[...], 9),
    ], axis=1)
    feats = jnp.concatenate([qi, side], axis=1)

    dot = functools.partial(jnp.dot, preferred_element_type=jnp.float32)
    x = _bn_relu(dot(feats, D1_ref[...]) + db1_ref[...], g1_ref[...], bb1_ref[...])
    x = _bn_relu(dot(x, D2_ref[...]) + db2_ref[...], g2_ref[...], bb2_ref[...])
    x = _bn_relu(dot(x, D3_ref[...]) + db3_ref[...], g3_ref[...], bb3_ref[...])
    out_ref[...] = dot(x, D4_ref[...]) + db4_ref[...]


def kernel(cand_video_id, cand_author_id, cand_video_type, cand_tag, tab,
           user_active_degree, follow_user_num_range, hist_video_id,
           hist_author_id, hist_mask, video_emb, author_emb, vt_emb, tag_emb,
           tab_emb, uad_emb, fur_emb, W1, b1, a1, W2, b2, a2, W3, b3,
           D1, db1, g1, bb1, D2, db2, g2, bb2, D3, db3, g3, bb3, D4, db4):
    i32 = jnp.int32

    author64 = jnp.concatenate(
        [author_emb, jnp.zeros((AUT_BUCKETS, 32), jnp.float32)], axis=1)
    q, keys, maskT = _build_sc_gather()(
        video_emb, author64,
        cand_video_id.astype(i32), cand_author_id.astype(i32),
        hist_video_id.astype(i32), hist_author_id.astype(i32),
        hist_mask.astype(i32))

    bB = 128
    grid = (B // bB,)
    full = lambda shape: pl.BlockSpec(shape, lambda i: tuple(0 for _ in shape))
    qi = pl.pallas_call(
        _att_body,
        grid=grid,
        in_specs=[
            pl.BlockSpec((bB, 128), lambda i: (i, 0)),
            pl.BlockSpec((L, bB, 128), lambda i: (0, i, 0)),
            pl.BlockSpec((L, bB), lambda i: (0, i)),
            full((384, 80)), full((1, 80)), full((1, 1)),
            full((80, 40)), full((1, 40)), full((1, 1)),
            full((40, 1)), full((1, 1)),
        ],
        out_specs=pl.BlockSpec((bB, 192), lambda i: (i, 0)),
        out_shape=jax.ShapeDtypeStruct((B, 192), jnp.float32),
    )(q, keys.reshape(L, B, 128), maskT,
      W1, b1.reshape(1, 80), a1.reshape(1, 1),
      W2, b2.reshape(1, 40), a2.reshape(1, 1),
      W3, b3.reshape(1, 1))

    logits = pl.pallas_call(
        _dnn_body,
        out_shape=jax.ShapeDtypeStruct((B, 1), jnp.float32),
    )(qi,
      cand_video_type.astype(i32).reshape(B, 1),
      cand_tag.astype(i32).reshape(B, 1),
      tab.astype(i32).reshape(B, 1),
      user_active_degree.astype(i32).reshape(B, 1),
      follow_user_num_range.astype(i32).reshape(B, 1),
      vt_emb, tag_emb, tab_emb, uad_emb, fur_emb,
      D1, db1.reshape(1, 256), g1.reshape(1, 256), bb1.reshape(1, 256),
      D2, db2.reshape(1, 128), g2.reshape(1, 128), bb2.reshape(1, 128),
      D3, db3.reshape(1, 64), g3.reshape(1, 64), bb3.reshape(1, 64),
      D4, db4.reshape(1, 1))
    return logits[:, 0]


# speedup vs baseline: 1.0133x; 1.0133x over previous
import functools

import jax
import jax.numpy as jnp
from jax import lax
from jax.experimental import pallas as pl
from jax.experimental.pallas import tpu as pltpu
from jax.experimental.pallas import tpu_sc as plsc

B = 4096
L = 50
VID_BUCKETS = 1000000
AUT_BUCKETS = 100000

NW = 32
CHUNK = 128


def _hash16(x, num_buckets):
    return jnp.where(x == 0, 0, lax.rem(x, num_buckets - 1) + 1)


@functools.lru_cache(maxsize=1)
def _build_sc_gather():
    mesh = plsc.VectorSubcoreMesh(core_axis_name="c", subcore_axis_name="s")

    @functools.partial(
        pl.kernel,
        mesh=mesh,
        out_type=[
            jax.ShapeDtypeStruct((B, 128), jnp.float32),
            jax.ShapeDtypeStruct((L * B, 128), jnp.float32),
            jax.ShapeDtypeStruct((L, B), jnp.int32),
        ],
        scratch_types=[
            pltpu.VMEM((CHUNK, L), jnp.int32),
            pltpu.VMEM((CHUNK, L), jnp.int32),
            pltpu.VMEM((CHUNK, L), jnp.int32),
            pltpu.VMEM((CHUNK,), jnp.int32),
            pltpu.VMEM((CHUNK,), jnp.int32),
            pltpu.VMEM((1, CHUNK), jnp.int32),
            pltpu.VMEM((CHUNK, 128), jnp.float32),
            pltpu.VMEM((CHUNK, 128), jnp.float32),
            pltpu.SemaphoreType.DMA,
            pltpu.SemaphoreType.DMA,
        ],
        compiler_params=pltpu.CompilerParams(use_tc_tiling_on_sc=False,
                                             needs_layout_passes=False),
    )
    def _sc_gather(video_hbm, author_hbm, cvid_hbm, caid_hbm, hvid_hbm,
                   haid_hbm, mask_hbm, out_q, out_keys, out_maskT,
                   idxv2, idxa2, msk2, chv, cha, mcol, bufv, bufa,
                   semv, sema):
        _sc_gather_body(video_hbm, author_hbm, cvid_hbm, caid_hbm, hvid_hbm,
                        haid_hbm, mask_hbm, out_q, out_keys, out_maskT,
                        idxv2, idxa2, msk2, chv, cha, mcol, bufv, bufa,
                        semv, sema)

    return _sc_gather


def _sc_gather_body(video_hbm, author_hbm, cvid_hbm, caid_hbm, hvid_hbm,
                    haid_hbm, mask_hbm, out_q, out_keys, out_maskT,
                    idxv2, idxa2, msk2, chv, cha, mcol, bufv, bufa,
                    semv, sema):
    wid = lax.axis_index("s") * 2 + lax.axis_index("c")
    b0 = pl.multiple_of(wid * CHUNK, 8)

    pltpu.sync_copy(cvid_hbm.at[pl.ds(b0, CHUNK)], chv)
    pltpu.sync_copy(caid_hbm.at[pl.ds(b0, CHUNK)], cha)
    for r in range(CHUNK // 16):
        s = pl.ds(r * 16, 16)
        chv[s] = _hash16(chv[s], VID_BUCKETS)
        cha[s] = _hash16(cha[s], AUT_BUCKETS)
    cpv = pltpu.async_copy(video_hbm.at[chv], bufv, semv)
    cpa = pltpu.async_copy(author_hbm.at[cha], bufa, sema)
    cpv.wait()
    pltpu.sync_copy(bufv.at[:, pl.ds(0, 64)],
                    out_q.at[pl.ds(b0, CHUNK), pl.ds(0, 64)])
    cpa.wait()
    pltpu.sync_copy(bufa.at[:, pl.ds(0, 64)],
                    out_q.at[pl.ds(b0, CHUNK), pl.ds(64, 64)])

    pltpu.sync_copy(hvid_hbm.at[pl.ds(b0, CHUNK), :], idxv2)
    pltpu.sync_copy(haid_hbm.at[pl.ds(b0, CHUNK), :], idxa2)
    pltpu.sync_copy(mask_hbm.at[pl.ds(b0, CHUNK), :], msk2)

    def body(l, carry):
        lvec = jnp.full((16,), 0, jnp.int32) + l
        for c in range(CHUNK // 16):
            rows = lax.iota(jnp.int32, 16) + (c * 16)
            s = pl.ds(c * 16, 16)
            chv[s] = _hash16(plsc.load_gather(idxv2, [rows, lvec]), VID_BUCKETS)
            cha[s] = _hash16(plsc.load_gather(idxa2, [rows, lvec]), AUT_BUCKETS)
            mcol[0, s] = plsc.load_gather(msk2, [rows, lvec])
        cpv2 = pltpu.async_copy(video_hbm.at[chv], bufv, semv)
        cpa2 = pltpu.async_copy(author_hbm.at[cha], bufa, sema)
        rbase = pl.multiple_of(l * B + b0, 8)
        cpv2.wait()
        pltpu.sync_copy(bufv.at[:, pl.ds(0, 64)],
                        out_keys.at[pl.ds(rbase, CHUNK), pl.ds(0, 64)])
        cpa2.wait()
        pltpu.sync_copy(bufa.at[:, pl.ds(0, 64)],
                        out_keys.at[pl.ds(rbase, CHUNK), pl.ds(64, 64)])
        pltpu.sync_copy(mcol, out_maskT.at[pl.ds(l, 1), pl.ds(b0, CHUNK)])
        return carry

    lax.fori_loop(0, L, body, 0, unroll=False)


def _prelu(x, a):
    return jnp.where(x >= 0, x, a * x)


def _pad_rows(w, total):
    return jnp.concatenate(
        [w, jnp.zeros((total - w.shape[0], w.shape[1]), jnp.float32)], axis=0)


def _att_body(q_ref, keys_ref, mask_ref,
              W1_ref, b1_ref, a1_ref, W2_ref, b2_ref, a2_ref, W3_ref, b3_ref,
              out_ref):
    bB = q_ref.shape[0]
    q = q_ref[...]
    keys = keys_ref[...]
    mask = mask_ref[...]

    W1 = W1_ref[...]
    Wa = W1[0:96, :]
    Wb = W1[96:192, :]
    Wc = W1[192:288, :]
    Wd = W1[288:384, :]
    Wq = _pad_rows(Wa + Wc, 128)
    Wk = _pad_rows(Wb - Wc, 128)
    Wp = _pad_rows(Wd, 128)
    b1 = b1_ref[...]
    a1 = a1_ref[0, 0]
    W2 = W2_ref[...]
    b2 = b2_ref[...]
    a2 = a2_ref[0, 0]
    W3 = W3_ref[...]
    b3 = b3_ref[0, 0]

    dot = functools.partial(jnp.dot, preferred_element_type=jnp.float32)

    kf = keys.reshape(L * bB, 128)
    term_q = dot(q, Wq)
    term_k = dot(kf, Wk)
    pf = (keys * q[None, :, :]).reshape(L * bB, 128)
    term_p = dot(pf, Wp)

    h = term_k + term_p + jnp.broadcast_to(
        term_q[None, :, :], (L, bB, 80)).reshape(L * bB, 80)
    h = _prelu(h + b1, a1)
    h = _prelu(dot(h, W2) + b2, a2)
    scores = dot(h, W3).reshape(L, bB) + b3

    neg = jnp.float32(-10000.0)
    scores = jnp.where(mask == 0, neg, scores)
    m = jnp.max(scores, axis=0, keepdims=True)
    e = jnp.exp(scores - m)
    w = e / jnp.sum(e, axis=0, keepdims=True)
    w = jnp.where(mask == 0, jnp.float32(0.0), w)

    interest = jnp.sum(keys * w[:, :, None], axis=0)

    out_ref[...] = jnp.concatenate(
        [q[:, 0:96], interest[:, 0:96]], axis=1)


def _onehot_lookup(idx2d, table, n):
    oh = jnp.where(
        idx2d == lax.broadcasted_iota(jnp.int32, (idx2d.shape[0], n), 1),
        jnp.float32(1.0), jnp.float32(0.0))
    return jnp.dot(oh, table, preferred_element_type=jnp.float32)


def _bn_relu(x, g, bb):
    m = jnp.mean(x, axis=0, keepdims=True)
    v = jnp.mean((x - m) ** 2, axis=0, keepdims=True)
    return jnp.maximum(g * (x - m) / jnp.sqrt(v + 1e-5) + bb, 0.0)


def _dnn_body(qi_ref, vt_i_ref, tag_i_ref, tab_i_ref, uad_i_ref, fur_i_ref,
              vt_ref, tag_ref, tab_ref, uad_ref, fur_ref,
              D1_ref, db1_ref, g1_ref, bb1_ref,
              D2_ref, db2_ref, g2_ref, bb2_ref,
              D3_ref, db3_ref, g3_ref, bb3_ref,
              D4_ref, db4_ref, out_ref):
    qi = qi_ref[...]
    side = jnp.concatenate([
        _onehot_lookup(vt_i_ref[...], vt_ref[...], 5),
        _onehot_lookup(tag_i_ref[...], tag_ref[...], 80),
        _onehot_lookup(tab_i_ref[...], tab_ref[...], 10),
        _onehot_lookup(uad_i_ref[...], uad_ref[...], 8),
        _onehot_lookup(fur_i_ref[...], fur_ref[...], 9),
    ], axis=1)
    feats = jnp.concatenate([qi, side], axis=1)

    dot = functools.partial(jnp.dot, preferred_element_type=jnp.float32)
    x = _bn_relu(dot(feats, D1_ref[...]) + db1_ref[...], g1_ref[...], bb1_ref[...])
    x = _bn_relu(dot(x, D2_ref[...]) + db2_ref[...], g2_ref[...], bb2_ref[...])
    x = _bn_relu(dot(x, D3_ref[...]) + db3_ref[...], g3_ref[...], bb3_ref[...])
    out_ref[...] = dot(x, D4_ref[...]) + db4_ref[...]


def kernel(cand_video_id, cand_author_id, cand_video_type, cand_tag, tab,
           user_active_degree, follow_user_num_range, hist_video_id,
           hist_author_id, hist_mask, video_emb, author_emb, vt_emb, tag_emb,
           tab_emb, uad_emb, fur_emb, W1, b1, a1, W2, b2, a2, W3, b3,
           D1, db1, g1, bb1, D2, db2, g2, bb2, D3, db3, g3, bb3, D4, db4):
    i32 = jnp.int32

    video128 = jnp.concatenate(
        [video_emb, jnp.zeros((VID_BUCKETS, 64), jnp.float32)], axis=1)
    author128 = jnp.concatenate(
        [author_emb, jnp.zeros((AUT_BUCKETS, 96), jnp.float32)], axis=1)
    q, keys, maskT = _build_sc_gather()(
        video128, author128,
        cand_video_id.astype(i32), cand_author_id.astype(i32),
        hist_video_id.astype(i32), hist_author_id.astype(i32),
        hist_mask.astype(i32))

    bB = 128
    grid = (B // bB,)
    full = lambda shape: pl.BlockSpec(shape, lambda i: tuple(0 for _ in shape))
    qi = pl.pallas_call(
        _att_body,
        grid=grid,
        in_specs=[
            pl.BlockSpec((bB, 128), lambda i: (i, 0)),
            pl.BlockSpec((L, bB, 128), lambda i: (0, i, 0)),
            pl.BlockSpec((L, bB), lambda i: (0, i)),
            full((384, 80)), full((1, 80)), full((1, 1)),
            full((80, 40)), full((1, 40)), full((1, 1)),
            full((40, 1)), full((1, 1)),
        ],
        out_specs=pl.BlockSpec((bB, 192), lambda i: (i, 0)),
        out_shape=jax.ShapeDtypeStruct((B, 192), jnp.float32),
    )(q, keys.reshape(L, B, 128), maskT,
      W1, b1.reshape(1, 80), a1.reshape(1, 1),
      W2, b2.reshape(1, 40), a2.reshape(1, 1),
      W3, b3.reshape(1, 1))

    logits = pl.pallas_call(
        _dnn_body,
        out_shape=jax.ShapeDtypeStruct((B, 1), jnp.float32),
    )(qi,
      cand_video_type.astype(i32).reshape(B, 1),
      cand_tag.astype(i32).reshape(B, 1),
      tab.astype(i32).reshape(B, 1),
      user_active_degree.astype(i32).reshape(B, 1),
      follow_user_num_range.astype(i32).reshape(B, 1),
      vt_emb, tag_emb, tab_emb, uad_emb, fur_emb,
      D1, db1.reshape(1, 256), g1.reshape(1, 256), bb1.reshape(1, 256),
      D2, db2.reshape(1, 128), g2.reshape(1, 128), bb2.reshape(1, 128),
      D3, db3.reshape(1, 64), g3.reshape(1, 64), bb3.reshape(1, 64),
      D4, db4.reshape(1, 1))
    return logits[:, 0]


# speedup vs baseline: 1.0559x; 1.0420x over previous
import functools

import jax
import jax.numpy as jnp
from jax import lax
from jax.experimental import pallas as pl
from jax.experimental.pallas import tpu as pltpu
from jax.experimental.pallas import tpu_sc as plsc

B = 4096
L = 50
VID_BUCKETS = 1000000
AUT_BUCKETS = 100000

NW = 32
CHUNK = 128


def _hash16(x, num_buckets):
    return jnp.where(x == 0, 0, lax.rem(x, num_buckets - 1) + 1)


@functools.lru_cache(maxsize=1)
def _build_sc_gather():
    mesh = plsc.VectorSubcoreMesh(core_axis_name="c", subcore_axis_name="s")

    @functools.partial(
        pl.kernel,
        mesh=mesh,
        out_type=[
            jax.ShapeDtypeStruct((B, 128), jnp.float32),
            jax.ShapeDtypeStruct((L * B, 128), jnp.float32),
            jax.ShapeDtypeStruct((L, B), jnp.int32),
        ],
        scratch_types=[
            pltpu.VMEM((CHUNK, L), jnp.int32),
            pltpu.VMEM((CHUNK, L), jnp.int32),
            pltpu.VMEM((CHUNK, L), jnp.int32),
            pltpu.VMEM((CHUNK,), jnp.int32),
            pltpu.VMEM((CHUNK,), jnp.int32),
            pltpu.VMEM((CHUNK,), jnp.int32),
            pltpu.VMEM((CHUNK,), jnp.int32),
            pltpu.VMEM((1, CHUNK), jnp.int32),
            pltpu.VMEM((CHUNK, 64), jnp.float32),
            pltpu.VMEM((CHUNK, 32), jnp.float32),
            pltpu.VMEM((CHUNK, 64), jnp.float32),
            pltpu.VMEM((CHUNK, 32), jnp.float32),
            pltpu.SemaphoreType.DMA,
            pltpu.SemaphoreType.DMA,
            pltpu.SemaphoreType.DMA,
            pltpu.SemaphoreType.DMA,
        ],
        compiler_params=pltpu.CompilerParams(use_tc_tiling_on_sc=False,
                                             needs_layout_passes=False),
    )
    def _sc_gather(video_hbm, author_hbm, cvid_hbm, caid_hbm, hvid_hbm,
                   haid_hbm, mask_hbm, out_q, out_keys, out_maskT,
                   idxv2, idxa2, msk2, chv0, cha0, chv1, cha1, mcol,
                   bufv0, bufa0, bufv1, bufa1,
                   semv0, sema0, semv1, sema1):
        _sc_gather_body(video_hbm, author_hbm, cvid_hbm, caid_hbm, hvid_hbm,
                        haid_hbm, mask_hbm, out_q, out_keys, out_maskT,
                        idxv2, idxa2, msk2, chv0, cha0, chv1, cha1, mcol,
                        bufv0, bufa0, bufv1, bufa1,
                        semv0, sema0, semv1, sema1)

    return _sc_gather


def _sc_gather_body(video_hbm, author_hbm, cvid_hbm, caid_hbm, hvid_hbm,
                    haid_hbm, mask_hbm, out_q, out_keys, out_maskT,
                    idxv2, idxa2, msk2, chv0, cha0, chv1, cha1, mcol,
                    bufv0, bufa0, bufv1, bufa1,
                    semv0, sema0, semv1, sema1):
    wid = lax.axis_index("s") * 2 + lax.axis_index("c")
    b0 = pl.multiple_of(wid * CHUNK, 8)

    pltpu.sync_copy(cvid_hbm.at[pl.ds(b0, CHUNK)], chv0)
    pltpu.sync_copy(caid_hbm.at[pl.ds(b0, CHUNK)], cha0)
    for r in range(CHUNK // 16):
        s = pl.ds(r * 16, 16)
        chv0[s] = _hash16(chv0[s], VID_BUCKETS)
        cha0[s] = _hash16(cha0[s], AUT_BUCKETS)
    cpv = pltpu.async_copy(video_hbm.at[chv0], bufv0, semv0)
    cpa = pltpu.async_copy(author_hbm.at[cha0], bufa0, sema0)
    pltpu.sync_copy(hvid_hbm.at[pl.ds(b0, CHUNK), :], idxv2)
    pltpu.sync_copy(haid_hbm.at[pl.ds(b0, CHUNK), :], idxa2)
    pltpu.sync_copy(mask_hbm.at[pl.ds(b0, CHUNK), :], msk2)
    cpv.wait()
    pltpu.sync_copy(bufv0, out_q.at[pl.ds(b0, CHUNK), pl.ds(0, 64)])
    cpa.wait()
    pltpu.sync_copy(bufa0, out_q.at[pl.ds(b0, CHUNK), pl.ds(64, 32)])

    def hash_col(l, chv, cha):
        lvec = jnp.full((16,), 0, jnp.int32) + l
        for c in range(CHUNK // 16):
            rows = lax.iota(jnp.int32, 16) + (c * 16)
            s = pl.ds(c * 16, 16)
            chv[s] = _hash16(plsc.load_gather(idxv2, [rows, lvec]), VID_BUCKETS)
            cha[s] = _hash16(plsc.load_gather(idxa2, [rows, lvec]), AUT_BUCKETS)

    def mask_col(l):
        lvec = jnp.full((16,), 0, jnp.int32) + l
        for c in range(CHUNK // 16):
            rows = lax.iota(jnp.int32, 16) + (c * 16)
            mcol[0, pl.ds(c * 16, 16)] = plsc.load_gather(msk2, [rows, lvec])
        pltpu.sync_copy(mcol, out_maskT.at[pl.ds(l, 1), pl.ds(b0, CHUNK)])

    def fire(chv, cha, bufv, bufa, semv, sema):
        pltpu.async_copy(video_hbm.at[chv], bufv, semv)
        pltpu.async_copy(author_hbm.at[cha], bufa, sema)

    def drain_store(chv, cha, bufv, bufa, semv, sema, l):
        rbase = pl.multiple_of(l * B + b0, 8)
        pltpu.make_async_copy(video_hbm.at[chv], bufv, semv).wait()
        pltpu.sync_copy(bufv, out_keys.at[pl.ds(rbase, CHUNK), pl.ds(0, 64)])
        pltpu.make_async_copy(author_hbm.at[cha], bufa, sema).wait()
        pltpu.sync_copy(bufa, out_keys.at[pl.ds(rbase, CHUNK), pl.ds(64, 32)])

    hash_col(0, chv0, cha0)
    fire(chv0, cha0, bufv0, bufa0, semv0, sema0)

    def body(j, carry):
        l0 = j * 2
        l1 = l0 + 1
        hash_col(l1, chv1, cha1)
        fire(chv1, cha1, bufv1, bufa1, semv1, sema1)
        mask_col(l0)
        drain_store(chv0, cha0, bufv0, bufa0, semv0, sema0, l0)

        @pl.when(j < (L // 2) - 1)
        def _():
            hash_col(l0 + 2, chv0, cha0)
            fire(chv0, cha0, bufv0, bufa0, semv0, sema0)

        mask_col(l1)
        drain_store(chv1, cha1, bufv1, bufa1, semv1, sema1, l1)
        return carry

    lax.fori_loop(0, L // 2, body, 0, unroll=False)


def _prelu(x, a):
    return jnp.where(x >= 0, x, a * x)


def _pad_rows(w, total):
    return jnp.concatenate(
        [w, jnp.zeros((total - w.shape[0], w.shape[1]), jnp.float32)], axis=0)


def _att_body(q_ref, keys_ref, mask_ref,
              W1_ref, b1_ref, a1_ref, W2_ref, b2_ref, a2_ref, W3_ref, b3_ref,
              out_ref):
    bB = q_ref.shape[0]
    lane = lax.broadcasted_iota(jnp.int32, (1, 128), 1)
    q = jnp.where(lane < 96, q_ref[...], 0.0)
    keys = jnp.where(lane[None, :, :] < 96, keys_ref[...], 0.0)
    mask = mask_ref[...]

    W1 = W1_ref[...]
    Wa = W1[0:96, :]
    Wb = W1[96:192, :]
    Wc = W1[192:288, :]
    Wd = W1[288:384, :]
    Wq = _pad_rows(Wa + Wc, 128)
    Wk = _pad_rows(Wb - Wc, 128)
    Wp = _pad_rows(Wd, 128)
    b1 = b1_ref[...]
    a1 = a1_ref[0, 0]
    W2 = W2_ref[...]
    b2 = b2_ref[...]
    a2 = a2_ref[0, 0]
    W3 = W3_ref[...]
    b3 = b3_ref[0, 0]

    dot = functools.partial(jnp.dot, preferred_element_type=jnp.float32)

    kf = keys.reshape(L * bB, 128)
    term_q = dot(q, Wq)
    term_k = dot(kf, Wk)
    pf = (keys * q[None, :, :]).reshape(L * bB, 128)
    term_p = dot(pf, Wp)

    h = term_k + term_p + jnp.broadcast_to(
        term_q[None, :, :], (L, bB, 80)).reshape(L * bB, 80)
    h = _prelu(h + b1, a1)
    h = _prelu(dot(h, W2) + b2, a2)
    scores = dot(h, W3).reshape(L, bB) + b3

    neg = jnp.float32(-10000.0)
    scores = jnp.where(mask == 0, neg, scores)
    m = jnp.max(scores, axis=0, keepdims=True)
    e = jnp.exp(scores - m)
    w = e / jnp.sum(e, axis=0, keepdims=True)
    w = jnp.where(mask == 0, jnp.float32(0.0), w)

    interest = jnp.sum(keys * w[:, :, None], axis=0)

    out_ref[...] = jnp.concatenate(
        [q[:, 0:96], interest[:, 0:96]], axis=1)


def _onehot_lookup(idx2d, table, n):
    oh = jnp.where(
        idx2d == lax.broadcasted_iota(jnp.int32, (idx2d.shape[0], n), 1),
        jnp.float32(1.0), jnp.float32(0.0))
    return jnp.dot(oh, table, preferred_element_type=jnp.float32)


def _bn_relu(x, g, bb):
    m = jnp.mean(x, axis=0, keepdims=True)
    v = jnp.mean((x - m) ** 2, axis=0, keepdims=True)
    return jnp.maximum(g * (x - m) / jnp.sqrt(v + 1e-5) + bb, 0.0)


def _dnn_body(qi_ref, vt_i_ref, tag_i_ref, tab_i_ref, uad_i_ref, fur_i_ref,
              vt_ref, tag_ref, tab_ref, uad_ref, fur_ref,
              D1_ref, db1_ref, g1_ref, bb1_ref,
              D2_ref, db2_ref, g2_ref, bb2_ref,
              D3_ref, db3_ref, g3_ref, bb3_ref,
              D4_ref, db4_ref, out_ref):
    qi = qi_ref[...]
    side = jnp.concatenate([
        _onehot_lookup(vt_i_ref[...], vt_ref[...], 5),
        _onehot_lookup(tag_i_ref[...], tag_ref[...], 80),
        _onehot_lookup(tab_i_ref[...], tab_ref[...], 10),
        _onehot_lookup(uad_i_ref[...], uad_ref[...], 8),
        _onehot_lookup(fur_i_ref[...], fur_ref[...], 9),
    ], axis=1)
    feats = jnp.concatenate([qi, side], axis=1)

    dot = functools.partial(jnp.dot, preferred_element_type=jnp.float32)
    x = _bn_relu(dot(feats, D1_ref[...]) + db1_ref[...], g1_ref[...], bb1_ref[...])
    x = _bn_relu(dot(x, D2_ref[...]) + db2_ref[...], g2_ref[...], bb2_ref[...])
    x = _bn_relu(dot(x, D3_ref[...]) + db3_ref[...], g3_ref[...], bb3_ref[...])
    out_ref[...] = dot(x, D4_ref[...]) + db4_ref[...]


def kernel(cand_video_id, cand_author_id, cand_video_type, cand_tag, tab,
           user_active_degree, follow_user_num_range, hist_video_id,
           hist_author_id, hist_mask, video_emb, author_emb, vt_emb, tag_emb,
           tab_emb, uad_emb, fur_emb, W1, b1, a1, W2, b2, a2, W3, b3,
           D1, db1, g1, bb1, D2, db2, g2, bb2, D3, db3, g3, bb3, D4, db4):
    i32 = jnp.int32

    q, keys, maskT = _build_sc_gather()(
        video_emb, author_emb,
        cand_video_id.astype(i32), cand_author_id.astype(i32),
        hist_video_id.astype(i32), hist_author_id.astype(i32),
        hist_mask.astype(i32))

    bB = 128
    grid = (B // bB,)
    full = lambda shape: pl.BlockSpec(shape, lambda i: tuple(0 for _ in shape))
    qi = pl.pallas_call(
        _att_body,
        grid=grid,
        in_specs=[
            pl.BlockSpec((bB, 128), lambda i: (i, 0)),
            pl.BlockSpec((L, bB, 128), lambda i: (0, i, 0)),
            pl.BlockSpec((L, bB), lambda i: (0, i)),
            full((384, 80)), full((1, 80)), full((1, 1)),
            full((80, 40)), full((1, 40)), full((1, 1)),
            full((40, 1)), full((1, 1)),
        ],
        out_specs=pl.BlockSpec((bB, 192), lambda i: (i, 0)),
        out_shape=jax.ShapeDtypeStruct((B, 192), jnp.float32),
    )(q, keys.reshape(L, B, 128), maskT,
      W1, b1.reshape(1, 80), a1.reshape(1, 1),
      W2, b2.reshape(1, 40), a2.reshape(1, 1),
      W3, b3.reshape(1, 1))

    logits = pl.pallas_call(
        _dnn_body,
        out_shape=jax.ShapeDtypeStruct((B, 1), jnp.float32),
    )(qi,
      cand_video_type.astype(i32).reshape(B, 1),
      cand_tag.astype(i32).reshape(B, 1),
      tab.astype(i32).reshape(B, 1),
      user_active_degree.astype(i32).reshape(B, 1),
      follow_user_num_range.astype(i32).reshape(B, 1),
      vt_emb, tag_emb, tab_emb, uad_emb, fur_emb,
      D1, db1.reshape(1, 256), g1.reshape(1, 256), bb1.reshape(1, 256),
      D2, db2.reshape(1, 128), g2.reshape(1, 128), bb2.reshape(1, 128),
      D3, db3.reshape(1, 64), g3.reshape(1, 64), bb3.reshape(1, 64),
      D4, db4.reshape(1, 1))
    return logits[:, 0]


# speedup vs baseline: 1.0632x; 1.0069x over previous
import functools

import jax
import jax.numpy as jnp
from jax import lax
from jax.experimental import pallas as pl
from jax.experimental.pallas import tpu as pltpu
from jax.experimental.pallas import tpu_sc as plsc

B = 4096
L = 50
VID_BUCKETS = 1000000
AUT_BUCKETS = 100000

NW = 32
CHUNK = 128


def _hash16(x, num_buckets):
    return jnp.where(x == 0, 0, lax.rem(x, num_buckets - 1) + 1)


@functools.lru_cache(maxsize=1)
def _build_sc_gather():
    mesh = plsc.VectorSubcoreMesh(core_axis_name="c", subcore_axis_name="s")

    @functools.partial(
        pl.kernel,
        mesh=mesh,
        out_type=[
            jax.ShapeDtypeStruct((B, 128), jnp.float32),
            jax.ShapeDtypeStruct((L * B, 128), jnp.float32),
            jax.ShapeDtypeStruct((L, B), jnp.int32),
        ],
        scratch_types=[
            pltpu.VMEM((CHUNK, L), jnp.int32),
            pltpu.VMEM((CHUNK, L), jnp.int32),
            pltpu.VMEM((CHUNK, L), jnp.int32),
            pltpu.VMEM((CHUNK,), jnp.int32),
            pltpu.VMEM((CHUNK,), jnp.int32),
            pltpu.VMEM((CHUNK,), jnp.int32),
            pltpu.VMEM((CHUNK,), jnp.int32),
            pltpu.VMEM((1, CHUNK), jnp.int32),
            pltpu.VMEM((CHUNK, 64), jnp.float32),
            pltpu.VMEM((CHUNK, 32), jnp.float32),
            pltpu.VMEM((CHUNK, 64), jnp.float32),
            pltpu.VMEM((CHUNK, 32), jnp.float32),
            pltpu.SemaphoreType.DMA,
            pltpu.SemaphoreType.DMA,
            pltpu.SemaphoreType.DMA,
            pltpu.SemaphoreType.DMA,
        ],
        compiler_params=pltpu.CompilerParams(use_tc_tiling_on_sc=False,
                                             needs_layout_passes=False),
    )
    def _sc_gather(video_hbm, author_hbm, cvid_hbm, caid_hbm, hvid_hbm,
                   haid_hbm, mask_hbm, out_q, out_keys, out_maskT,
                   idxv2, idxa2, msk2, chv0, cha0, chv1, cha1, mcol,
                   bufv0, bufa0, bufv1, bufa1,
                   semv0, sema0, semv1, sema1):
        _sc_gather_body(video_hbm, author_hbm, cvid_hbm, caid_hbm, hvid_hbm,
                        haid_hbm, mask_hbm, out_q, out_keys, out_maskT,
                        idxv2, idxa2, msk2, chv0, cha0, chv1, cha1, mcol,
                        bufv0, bufa0, bufv1, bufa1,
                        semv0, sema0, semv1, sema1)

    return _sc_gather


def _sc_gather_body(video_hbm, author_hbm, cvid_hbm, caid_hbm, hvid_hbm,
                    haid_hbm, mask_hbm, out_q, out_keys, out_maskT,
                    idxv2, idxa2, msk2, chv0, cha0, chv1, cha1, mcol,
                    bufv0, bufa0, bufv1, bufa1,
                    semv0, sema0, semv1, sema1):
    wid = lax.axis_index("s") * 2 + lax.axis_index("c")
    b0 = pl.multiple_of(wid * CHUNK, 8)

    pltpu.sync_copy(cvid_hbm.at[pl.ds(b0, CHUNK)], chv0)
    pltpu.sync_copy(caid_hbm.at[pl.ds(b0, CHUNK)], cha0)
    for r in range(CHUNK // 16):
        s = pl.ds(r * 16, 16)
        chv0[s] = _hash16(chv0[s], VID_BUCKETS)
        cha0[s] = _hash16(cha0[s], AUT_BUCKETS)
    cpv = pltpu.async_copy(video_hbm.at[chv0], bufv0, semv0)
    cpa = pltpu.async_copy(author_hbm.at[cha0], bufa0, sema0)
    pltpu.sync_copy(hvid_hbm.at[pl.ds(b0, CHUNK), :], idxv2)
    pltpu.sync_copy(haid_hbm.at[pl.ds(b0, CHUNK), :], idxa2)
    pltpu.sync_copy(mask_hbm.at[pl.ds(b0, CHUNK), :], msk2)
    cpv.wait()
    pltpu.sync_copy(bufv0, out_q.at[pl.ds(b0, CHUNK), pl.ds(0, 64)])
    cpa.wait()
    pltpu.sync_copy(bufa0, out_q.at[pl.ds(b0, CHUNK), pl.ds(64, 32)])

    def hash_col(l, chv, cha):
        lvec = jnp.full((16,), 0, jnp.int32) + l
        for c in range(CHUNK // 16):
            rows = lax.iota(jnp.int32, 16) + (c * 16)
            s = pl.ds(c * 16, 16)
            chv[s] = _hash16(plsc.load_gather(idxv2, [rows, lvec]), VID_BUCKETS)
            cha[s] = _hash16(plsc.load_gather(idxa2, [rows, lvec]), AUT_BUCKETS)

    def mask_col(l):
        lvec = jnp.full((16,), 0, jnp.int32) + l
        for c in range(CHUNK // 16):
            rows = lax.iota(jnp.int32, 16) + (c * 16)
            mcol[0, pl.ds(c * 16, 16)] = plsc.load_gather(msk2, [rows, lvec])
        pltpu.sync_copy(mcol, out_maskT.at[pl.ds(l, 1), pl.ds(b0, CHUNK)])

    def fire(chv, cha, bufv, bufa, semv, sema):
        pltpu.async_copy(video_hbm.at[chv], bufv, semv)
        pltpu.async_copy(author_hbm.at[cha], bufa, sema)

    def drain_store(chv, cha, bufv, bufa, semv, sema, l):
        rbase = pl.multiple_of(l * B + b0, 8)
        pltpu.make_async_copy(video_hbm.at[chv], bufv, semv).wait()
        pltpu.sync_copy(bufv, out_keys.at[pl.ds(rbase, CHUNK), pl.ds(0, 64)])
        pltpu.make_async_copy(author_hbm.at[cha], bufa, sema).wait()
        pltpu.sync_copy(bufa, out_keys.at[pl.ds(rbase, CHUNK), pl.ds(64, 32)])

    hash_col(0, chv0, cha0)
    fire(chv0, cha0, bufv0, bufa0, semv0, sema0)

    def body(j, carry):
        l0 = j * 2
        l1 = l0 + 1
        hash_col(l1, chv1, cha1)
        fire(chv1, cha1, bufv1, bufa1, semv1, sema1)
        mask_col(l0)
        drain_store(chv0, cha0, bufv0, bufa0, semv0, sema0, l0)

        @pl.when(j < (L // 2) - 1)
        def _():
            hash_col(l0 + 2, chv0, cha0)
            fire(chv0, cha0, bufv0, bufa0, semv0, sema0)

        mask_col(l1)
        drain_store(chv1, cha1, bufv1, bufa1, semv1, sema1, l1)
        return carry

    lax.fori_loop(0, L // 2, body, 0, unroll=False)


def _prelu(x, a):
    return jnp.where(x >= 0, x, a * x)


def _pad_rows(w, total):
    return jnp.concatenate(
        [w, jnp.zeros((total - w.shape[0], w.shape[1]), jnp.float32)], axis=0)


def _att_body(q_ref, keys_ref, mask_ref,
              W1_ref, b1_ref, a1_ref, W2_ref, b2_ref, a2_ref, W3_ref, b3_ref,
              out_ref):
    bB = q_ref.shape[0]
    lane = lax.broadcasted_iota(jnp.int32, (1, 128), 1)
    q = jnp.where(lane < 96, q_ref[...], 0.0)
    keys = jnp.where(lane[None, :, :] < 96, keys_ref[...], 0.0)
    mask = mask_ref[...]

    W1 = W1_ref[...]
    Wa = W1[0:96, :]
    Wb = W1[96:192, :]
    Wc = W1[192:288, :]
    Wd = W1[288:384, :]
    Wq = _pad_rows(Wa + Wc, 128)
    Wk = _pad_rows(Wb - Wc, 128)
    Wp = _pad_rows(Wd, 128)
    b1 = b1_ref[...]
    a1 = a1_ref[0, 0]
    W2 = W2_ref[...]
    b2 = b2_ref[...]
    a2 = a2_ref[0, 0]
    W3 = W3_ref[...]
    b3 = b3_ref[0, 0]

    dot = functools.partial(jnp.dot, preferred_element_type=jnp.float32)

    kf = keys.reshape(L * bB, 128)
    term_q = dot(q, Wq)
    term_k = dot(kf, Wk)
    pf = (keys * q[None, :, :]).reshape(L * bB, 128)
    term_p = dot(pf, Wp)

    h = term_k + term_p + jnp.broadcast_to(
        term_q[None, :, :], (L, bB, 80)).reshape(L * bB, 80)
    h = _prelu(h + b1, a1)
    h = _prelu(dot(h, W2) + b2, a2)
    scores = dot(h, W3).reshape(L, bB) + b3

    neg = jnp.float32(-10000.0)
    scores = jnp.where(mask == 0, neg, scores)
    m = jnp.max(scores, axis=0, keepdims=True)
    e = jnp.exp(scores - m)
    w = e / jnp.sum(e, axis=0, keepdims=True)
    w = jnp.where(mask == 0, jnp.float32(0.0), w)

    interest = jnp.sum(keys * w[:, :, None], axis=0)

    out_ref[...] = jnp.concatenate(
        [q[:, 0:96], interest[:, 0:96]], axis=1)


def _onehot_lookup(idx2d, table, n):
    oh = jnp.where(
        idx2d == lax.broadcasted_iota(jnp.int32, (idx2d.shape[0], n), 1),
        jnp.float32(1.0), jnp.float32(0.0))
    return jnp.dot(oh, table, preferred_element_type=jnp.float32)


def _bn_relu(x, g, bb):
    m = jnp.mean(x, axis=0, keepdims=True)
    v = jnp.mean((x - m) ** 2, axis=0, keepdims=True)
    return jnp.maximum(g * (x - m) / jnp.sqrt(v + 1e-5) + bb, 0.0)


def _dnn_body(qi_ref, vt_i_ref, tag_i_ref, tab_i_ref, uad_i_ref, fur_i_ref,
              vt_ref, tag_ref, tab_ref, uad_ref, fur_ref,
              D1_ref, db1_ref, g1_ref, bb1_ref,
              D2_ref, db2_ref, g2_ref, bb2_ref,
              D3_ref, db3_ref, g3_ref, bb3_ref,
              D4_ref, db4_ref, out_ref):
    qi = qi_ref[...]
    side = jnp.concatenate([
        _onehot_lookup(vt_i_ref[...], vt_ref[...], 5),
        _onehot_lookup(tag_i_ref[...], tag_ref[...], 80),
        _onehot_lookup(tab_i_ref[...], tab_ref[...], 10),
        _onehot_lookup(uad_i_ref[...], uad_ref[...], 8),
        _onehot_lookup(fur_i_ref[...], fur_ref[...], 9),
    ], axis=1)
    feats = jnp.concatenate([qi, side], axis=1)

    dot = functools.partial(jnp.dot, preferred_element_type=jnp.float32)
    x = _bn_relu(dot(feats, D1_ref[...]) + db1_ref[...], g1_ref[...], bb1_ref[...])
    x = _bn_relu(dot(x, D2_ref[...]) + db2_ref[...], g2_ref[...], bb2_ref[...])
    x = _bn_relu(dot(x, D3_ref[...]) + db3_ref[...], g3_ref[...], bb3_ref[...])
    out_ref[...] = dot(x, D4_ref[...]) + db4_ref[...]


def kernel(cand_video_id, cand_author_id, cand_video_type, cand_tag, tab,
           user_active_degree, follow_user_num_range, hist_video_id,
           hist_author_id, hist_mask, video_emb, author_emb, vt_emb, tag_emb,
           tab_emb, uad_emb, fur_emb, W1, b1, a1, W2, b2, a2, W3, b3,
           D1, db1, g1, bb1, D2, db2, g2, bb2, D3, db3, g3, bb3, D4, db4):
    i32 = jnp.int32

    q, keys, maskT = _build_sc_gather()(
        video_emb, author_emb,
        cand_video_id.astype(i32), cand_author_id.astype(i32),
        hist_video_id.astype(i32), hist_author_id.astype(i32),
        hist_mask.astype(i32))

    bB = 256
    grid = (B // bB,)
    full = lambda shape: pl.BlockSpec(shape, lambda i: tuple(0 for _ in shape))
    qi = pl.pallas_call(
        _att_body,
        grid=grid,
        in_specs=[
            pl.BlockSpec((bB, 128), lambda i: (i, 0)),
            pl.BlockSpec((L, bB, 128), lambda i: (0, i, 0)),
            pl.BlockSpec((L, bB), lambda i: (0, i)),
            full((384, 80)), full((1, 80)), full((1, 1)),
            full((80, 40)), full((1, 40)), full((1, 1)),
            full((40, 1)), full((1, 1)),
        ],
        out_specs=pl.BlockSpec((bB, 192), lambda i: (i, 0)),
        out_shape=jax.ShapeDtypeStruct((B, 192), jnp.float32),
    )(q, keys.reshape(L, B, 128), maskT,
      W1, b1.reshape(1, 80), a1.reshape(1, 1),
      W2, b2.reshape(1, 40), a2.reshape(1, 1),
      W3, b3.reshape(1, 1))

    logits = pl.pallas_call(
        _dnn_body,
        out_shape=jax.ShapeDtypeStruct((B, 1), jnp.float32),
    )(qi,
      cand_video_type.astype(i32).reshape(B, 1),
      cand_tag.astype(i32).reshape(B, 1),
      tab.astype(i32).reshape(B, 1),
      user_active_degree.astype(i32).reshape(B, 1),
      follow_user_num_range.astype(i32).reshape(B, 1),
      vt_emb, tag_emb, tab_emb, uad_emb, fur_emb,
      D1, db1.reshape(1, 256), g1.reshape(1, 256), bb1.reshape(1, 256),
      D2, db2.reshape(1, 128), g2.reshape(1, 128), bb2.reshape(1, 128),
      D3, db3.reshape(1, 64), g3.reshape(1, 64), bb3.reshape(1, 64),
      D4, db4.reshape(1, 1))
    return logits[:, 0]


# speedup vs baseline: 1.0647x; 1.0014x over previous
import functools

import jax
import jax.numpy as jnp
from jax import lax
from jax.experimental import pallas as pl
from jax.experimental.pallas import tpu as pltpu
from jax.experimental.pallas import tpu_sc as plsc

B = 4096
L = 50
VID_BUCKETS = 1000000
AUT_BUCKETS = 100000

NW = 32
CHUNK = 128


def _hash16(x, num_buckets):
    return jnp.where(x == 0, 0, lax.rem(x, num_buckets - 1) + 1)


@functools.lru_cache(maxsize=1)
def _build_sc_gather():
    mesh = plsc.VectorSubcoreMesh(core_axis_name="c", subcore_axis_name="s")

    @functools.partial(
        pl.kernel,
        mesh=mesh,
        out_type=[
            jax.ShapeDtypeStruct((B, 128), jnp.float32),
            jax.ShapeDtypeStruct((L * B, 128), jnp.float32),
            jax.ShapeDtypeStruct((L, B), jnp.int32),
        ],
        scratch_types=[
            pltpu.VMEM((CHUNK, L), jnp.int32),
            pltpu.VMEM((CHUNK, L), jnp.int32),
            pltpu.VMEM((CHUNK, L), jnp.int32),
            pltpu.VMEM((CHUNK,), jnp.int32),
            pltpu.VMEM((CHUNK,), jnp.int32),
            pltpu.VMEM((CHUNK,), jnp.int32),
            pltpu.VMEM((CHUNK,), jnp.int32),
            pltpu.VMEM((1, CHUNK), jnp.int32),
            pltpu.VMEM((CHUNK, 64), jnp.float32),
            pltpu.VMEM((CHUNK, 32), jnp.float32),
            pltpu.VMEM((CHUNK, 64), jnp.float32),
            pltpu.VMEM((CHUNK, 32), jnp.float32),
            pltpu.SemaphoreType.DMA,
            pltpu.SemaphoreType.DMA,
            pltpu.SemaphoreType.DMA,
            pltpu.SemaphoreType.DMA,
        ],
        compiler_params=pltpu.CompilerParams(use_tc_tiling_on_sc=False,
                                             needs_layout_passes=False),
    )
    def _sc_gather(video_hbm, author_hbm, cvid_hbm, caid_hbm, hvid_hbm,
                   haid_hbm, mask_hbm, out_q, out_keys, out_maskT,
                   idxv2, idxa2, msk2, chv0, cha0, chv1, cha1, mcol,
                   bufv0, bufa0, bufv1, bufa1,
                   semv0, sema0, semv1, sema1):
        _sc_gather_body(video_hbm, author_hbm, cvid_hbm, caid_hbm, hvid_hbm,
                        haid_hbm, mask_hbm, out_q, out_keys, out_maskT,
                        idxv2, idxa2, msk2, chv0, cha0, chv1, cha1, mcol,
                        bufv0, bufa0, bufv1, bufa1,
                        semv0, sema0, semv1, sema1)

    return _sc_gather


def _sc_gather_body(video_hbm, author_hbm, cvid_hbm, caid_hbm, hvid_hbm,
                    haid_hbm, mask_hbm, out_q, out_keys, out_maskT,
                    idxv2, idxa2, msk2, chv0, cha0, chv1, cha1, mcol,
                    bufv0, bufa0, bufv1, bufa1,
                    semv0, sema0, semv1, sema1):
    wid = lax.axis_index("s") * 2 + lax.axis_index("c")
    b0 = pl.multiple_of(wid * CHUNK, 8)

    pltpu.sync_copy(cvid_hbm.at[pl.ds(b0, CHUNK)], chv0)
    pltpu.sync_copy(caid_hbm.at[pl.ds(b0, CHUNK)], cha0)
    for r in range(CHUNK // 16):
        s = pl.ds(r * 16, 16)
        chv0[s] = _hash16(chv0[s], VID_BUCKETS)
        cha0[s] = _hash16(cha0[s], AUT_BUCKETS)
    cpv = pltpu.async_copy(video_hbm.at[chv0], bufv0, semv0)
    cpa = pltpu.async_copy(author_hbm.at[cha0], bufa0, sema0)
    pltpu.sync_copy(hvid_hbm.at[pl.ds(b0, CHUNK), :], idxv2)
    pltpu.sync_copy(haid_hbm.at[pl.ds(b0, CHUNK), :], idxa2)
    pltpu.sync_copy(mask_hbm.at[pl.ds(b0, CHUNK), :], msk2)
    cpv.wait()
    pltpu.sync_copy(bufv0, out_q.at[pl.ds(b0, CHUNK), pl.ds(0, 64)])
    cpa.wait()
    pltpu.sync_copy(bufa0, out_q.at[pl.ds(b0, CHUNK), pl.ds(64, 32)])

    def hash_col(l, chv, cha):
        lvec = jnp.full((16,), 0, jnp.int32) + l
        for c in range(CHUNK // 16):
            rows = lax.iota(jnp.int32, 16) + (c * 16)
            s = pl.ds(c * 16, 16)
            chv[s] = _hash16(plsc.load_gather(idxv2, [rows, lvec]), VID_BUCKETS)
            cha[s] = _hash16(plsc.load_gather(idxa2, [rows, lvec]), AUT_BUCKETS)

    def mask_col(l):
        lvec = jnp.full((16,), 0, jnp.int32) + l
        for c in range(CHUNK // 16):
            rows = lax.iota(jnp.int32, 16) + (c * 16)
            mcol[0, pl.ds(c * 16, 16)] = plsc.load_gather(msk2, [rows, lvec])
        pltpu.sync_copy(mcol, out_maskT.at[pl.ds(l, 1), pl.ds(b0, CHUNK)])

    def fire(chv, cha, bufv, bufa, semv, sema):
        pltpu.async_copy(video_hbm.at[chv], bufv, semv)
        pltpu.async_copy(author_hbm.at[cha], bufa, sema)

    def drain_store(chv, cha, bufv, bufa, semv, sema, l):
        rbase = pl.multiple_of(l * B + b0, 8)
        pltpu.make_async_copy(video_hbm.at[chv], bufv, semv).wait()
        pltpu.sync_copy(bufv, out_keys.at[pl.ds(rbase, CHUNK), pl.ds(0, 64)])
        pltpu.make_async_copy(author_hbm.at[cha], bufa, sema).wait()
        pltpu.sync_copy(bufa, out_keys.at[pl.ds(rbase, CHUNK), pl.ds(64, 32)])

    hash_col(0, chv0, cha0)
    fire(chv0, cha0, bufv0, bufa0, semv0, sema0)

    def body(j, carry):
        l0 = j * 2
        l1 = l0 + 1
        hash_col(l1, chv1, cha1)
        fire(chv1, cha1, bufv1, bufa1, semv1, sema1)
        mask_col(l0)
        drain_store(chv0, cha0, bufv0, bufa0, semv0, sema0, l0)

        @pl.when(j < (L // 2) - 1)
        def _():
            hash_col(l0 + 2, chv0, cha0)
            fire(chv0, cha0, bufv0, bufa0, semv0, sema0)

        mask_col(l1)
        drain_store(chv1, cha1, bufv1, bufa1, semv1, sema1, l1)
        return carry

    lax.fori_loop(0, L // 2, body, 0, unroll=False)


def _prelu(x, a):
    return jnp.where(x >= 0, x, a * x)


def _pad_rows(w, total):
    return jnp.concatenate(
        [w, jnp.zeros((total - w.shape[0], w.shape[1]), jnp.float32)], axis=0)


def _att_body(q_ref, keys_ref, mask_ref,
              W1_ref, b1_ref, a1_ref, W2_ref, b2_ref, a2_ref, W3_ref, b3_ref,
              out_ref):
    bB = q_ref.shape[0]
    lane = lax.broadcasted_iota(jnp.int32, (1, 128), 1)
    q = jnp.where(lane < 96, q_ref[...], 0.0)
    keys = jnp.where(lane[None, :, :] < 96, keys_ref[...], 0.0)
    mask = mask_ref[...]

    W1 = W1_ref[...]
    Wa = W1[0:96, :]
    Wb = W1[96:192, :]
    Wc = W1[192:288, :]
    Wd = W1[288:384, :]
    Wq = _pad_rows(Wa + Wc, 128)
    Wk = _pad_rows(Wb - Wc, 128)
    Wp = _pad_rows(Wd, 128)
    b1 = b1_ref[...]
    a1 = a1_ref[0, 0]
    W2 = W2_ref[...]
    b2 = b2_ref[...]
    a2 = a2_ref[0, 0]
    W3 = W3_ref[...]
    b3 = b3_ref[0, 0]

    dot = functools.partial(jnp.dot, preferred_element_type=jnp.float32)

    kf = keys.reshape(L * bB, 128)
    term_q = dot(q, Wq)
    term_k = dot(kf, Wk)
    pf = (keys * q[None, :, :]).reshape(L * bB, 128)
    term_p = dot(pf, Wp)

    h = term_k + term_p + jnp.broadcast_to(
        term_q[None, :, :], (L, bB, 80)).reshape(L * bB, 80)
    h = _prelu(h + b1, a1)
    h = _prelu(dot(h, W2) + b2, a2)
    scores = dot(h, W3).reshape(L, bB) + b3

    neg = jnp.float32(-10000.0)
    scores = jnp.where(mask == 0, neg, scores)
    m = jnp.max(scores, axis=0, keepdims=True)
    e = jnp.exp(scores - m)
    w = e / jnp.sum(e, axis=0, keepdims=True)
    w = jnp.where(mask == 0, jnp.float32(0.0), w)

    interest = jnp.sum(keys * w[:, :, None], axis=0)

    out_ref[...] = jnp.concatenate(
        [q[:, 0:96], interest[:, 0:96]], axis=1)


def _onehot_lookup(idx2d, table, n):
    oh = jnp.where(
        idx2d == lax.broadcasted_iota(jnp.int32, (idx2d.shape[0], n), 1),
        jnp.float32(1.0), jnp.float32(0.0))
    return jnp.dot(oh, table, preferred_element_type=jnp.float32)


def _bn_relu(x, g, bb):
    m = jnp.mean(x, axis=0, keepdims=True)
    v = jnp.mean((x - m) ** 2, axis=0, keepdims=True)
    return jnp.maximum(g * (x - m) / jnp.sqrt(v + 1e-5) + bb, 0.0)


def _dnn_body(qi_ref, vt_i_ref, tag_i_ref, tab_i_ref, uad_i_ref, fur_i_ref,
              vt_ref, tag_ref, tab_ref, uad_ref, fur_ref,
              D1_ref, db1_ref, g1_ref, bb1_ref,
              D2_ref, db2_ref, g2_ref, bb2_ref,
              D3_ref, db3_ref, g3_ref, bb3_ref,
              D4_ref, db4_ref, out_ref):
    qi = qi_ref[...]
    side = jnp.concatenate([
        _onehot_lookup(vt_i_ref[...], vt_ref[...], 5),
        _onehot_lookup(tag_i_ref[...], tag_ref[...], 80),
        _onehot_lookup(tab_i_ref[...], tab_ref[...], 10),
        _onehot_lookup(uad_i_ref[...], uad_ref[...], 8),
        _onehot_lookup(fur_i_ref[...], fur_ref[...], 9),
    ], axis=1)
    feats = jnp.concatenate([qi, side], axis=1)

    dot = functools.partial(jnp.dot, preferred_element_type=jnp.float32)
    x = _bn_relu(dot(feats, D1_ref[...]) + db1_ref[...], g1_ref[...], bb1_ref[...])
    x = _bn_relu(dot(x, D2_ref[...]) + db2_ref[...], g2_ref[...], bb2_ref[...])
    x = _bn_relu(dot(x, D3_ref[...]) + db3_ref[...], g3_ref[...], bb3_ref[...])
    out_ref[...] = dot(x, D4_ref[...]) + db4_ref[...]


def kernel(cand_video_id, cand_author_id, cand_video_type, cand_tag, tab,
           user_active_degree, follow_user_num_range, hist_video_id,
           hist_author_id, hist_mask, video_emb, author_emb, vt_emb, tag_emb,
           tab_emb, uad_emb, fur_emb, W1, b1, a1, W2, b2, a2, W3, b3,
           D1, db1, g1, bb1, D2, db2, g2, bb2, D3, db3, g3, bb3, D4, db4):
    i32 = jnp.int32

    q, keys, maskT = _build_sc_gather()(
        video_emb, author_emb,
        cand_video_id.astype(i32), cand_author_id.astype(i32),
        hist_video_id.astype(i32), hist_author_id.astype(i32),
        hist_mask.astype(i32))

    bB = 512
    grid = (B // bB,)
    full = lambda shape: pl.BlockSpec(shape, lambda i: tuple(0 for _ in shape))
    qi = pl.pallas_call(
        _att_body,
        grid=grid,
        in_specs=[
            pl.BlockSpec((bB, 128), lambda i: (i, 0)),
            pl.BlockSpec((L, bB, 128), lambda i: (0, i, 0)),
            pl.BlockSpec((L, bB), lambda i: (0, i)),
            full((384, 80)), full((1, 80)), full((1, 1)),
            full((80, 40)), full((1, 40)), full((1, 1)),
            full((40, 1)), full((1, 1)),
        ],
        out_specs=pl.BlockSpec((bB, 192), lambda i: (i, 0)),
        out_shape=jax.ShapeDtypeStruct((B, 192), jnp.float32),
    )(q, keys.reshape(L, B, 128), maskT,
      W1, b1.reshape(1, 80), a1.reshape(1, 1),
      W2, b2.reshape(1, 40), a2.reshape(1, 1),
      W3, b3.reshape(1, 1))

    logits = pl.pallas_call(
        _dnn_body,
        out_shape=jax.ShapeDtypeStruct((B, 1), jnp.float32),
    )(qi,
      cand_video_type.astype(i32).reshape(B, 1),
      cand_tag.astype(i32).reshape(B, 1),
      tab.astype(i32).reshape(B, 1),
      user_active_degree.astype(i32).reshape(B, 1),
      follow_user_num_range.astype(i32).reshape(B, 1),
      vt_emb, tag_emb, tab_emb, uad_emb, fur_emb,
      D1, db1.reshape(1, 256), g1.reshape(1, 256), bb1.reshape(1, 256),
      D2, db2.reshape(1, 128), g2.reshape(1, 128), bb2.reshape(1, 128),
      D3, db3.reshape(1, 64), g3.reshape(1, 64), bb3.reshape(1, 64),
      D4, db4.reshape(1, 1))
    return logits[:, 0]
